# Initial kernel scaffold; baseline (speedup 1.0000x reference)
import jax
import jax.numpy as jnp
from jax.experimental import pallas as pl


def kernel(x, edge_index, W1, b1, W2, b2):
    raise NotImplementedError("write your pallas kernel here")


# speedup vs baseline: 5.4332x; 5.4332x over previous
import functools

import jax
import jax.numpy as jnp
from jax import lax
from jax.experimental import pallas as pl
from jax.experimental.pallas import tpu as pltpu
from jax.experimental.pallas import tpu_sc as plsc

N = 10000
D = 128
H = 128
C = 16
E = 320000
NC, NS, L = 2, 16, 16
NW = NC * NS
NPAD = 10240
RPT = NPAD // NS
EPAD = 327680
K = 128
BN = 1000


def _mesh():
    return plsc.VectorSubcoreMesh(
        core_axis_name="c", subcore_axis_name="s", num_cores=NC, num_subcores=NS
    )


def _deg_call(src_pd, dst_pd):
    epc = EPAD // NS
    nch = epc // K

    @functools.partial(
        pl.kernel,
        out_type=jax.ShapeDtypeStruct((2, NPAD), jnp.float32),
        mesh=_mesh(),
        scratch_types=[
            pltpu.VMEM((K,), jnp.int32),
            pltpu.VMEM((K,), jnp.float32),
            pltpu.VMEM((RPT,), jnp.float32),
            pltpu.VMEM_SHARED((NPAD,), jnp.float32),
        ],
    )
    def deg_kernel(src_ref, dst_ref, out_ref, idx, ones, zb, acc):
        c = lax.axis_index("c")
        s = lax.axis_index("s")
        one16 = jnp.ones((L,), jnp.float32)
        zero16 = jnp.zeros((L,), jnp.float32)
        for j in range(K // L):
            ones[pl.ds(j * L, L)] = one16

        def zb_body(j, carry):
            zb[pl.ds(pl.multiple_of(j * L, 8), L)] = zero16
            return carry

        lax.fori_loop(0, RPT // L, zb_body, 0)
        pltpu.sync_copy(zb, acc.at[pl.ds(pl.multiple_of(s * RPT, 8), RPT)])
        plsc.subcore_barrier()

        def run(ref):
            def body(i, carry):
                off = pl.multiple_of(s * epc + i * K, 8)
                pltpu.sync_copy(ref.at[pl.ds(off, K)], idx)
                pltpu.sync_copy(ones, acc.at[idx], add=True)
                return carry

            lax.fori_loop(0, nch, body, 0)

        @pl.when(c == 0)
        def _():
            run(src_ref)

        @pl.when(c == 1)
        def _():
            run(dst_ref)

        plsc.subcore_barrier()
        st = pl.multiple_of(s * RPT, 8)
        pltpu.sync_copy(acc.at[pl.ds(st, RPT)], out_ref.at[c, pl.ds(st, RPT)])

    return deg_kernel(src_pd, dst_pd)


def _agg_split_call(h2lay, src_pg, dst_pd):
    F = H // NC
    epc = EPAD // NS
    nch = epc // K
    cpr = F // L

    @functools.partial(
        pl.kernel,
        out_type=jax.ShapeDtypeStruct((NC, NPAD, F), jnp.float32),
        mesh=_mesh(),
        scratch_types=[
            pltpu.VMEM((K,), jnp.int32),
            pltpu.VMEM((K,), jnp.int32),
            pltpu.VMEM((K, F), jnp.float32),
            pltpu.VMEM((RPT, F), jnp.float32),
            pltpu.VMEM_SHARED((NPAD, F), jnp.float32),
            pltpu.SemaphoreType.DMA,
        ],
        compiler_params=pltpu.CompilerParams(use_tc_tiling_on_sc=False),
    )
    def agg_kernel(h_ref, src_ref, dst_ref, out_ref, sidx, didx, rows, zb, acc, sem):
        c = lax.axis_index("c")
        s = lax.axis_index("s")
        zero16 = jnp.zeros((L,), jnp.float32)

        def zb_body(j, carry):
            r = j // cpr
            col = (j % cpr) * L
            zb[r, pl.ds(col, L)] = zero16
            return carry

        lax.fori_loop(0, RPT * cpr, zb_body, 0)
        pltpu.sync_copy(zb, acc.at[pl.ds(pl.multiple_of(s * RPT, 8), RPT)])
        plsc.subcore_barrier()
        tab = h_ref.at[c]

        def body(i, carry):
            off = pl.multiple_of(s * epc + i * K, 8)
            pltpu.sync_copy(src_ref.at[pl.ds(off, K)], sidx)
            pltpu.sync_copy(dst_ref.at[pl.ds(off, K)], didx)
            pltpu.async_copy(tab.at[sidx], rows, sem).wait()
            pltpu.sync_copy(rows, acc.at[didx], add=True)
            return carry

        lax.fori_loop(0, nch, body, 0)
        plsc.subcore_barrier()
        st = pl.multiple_of(s * RPT, 8)
        pltpu.sync_copy(acc.at[pl.ds(st, RPT)], out_ref.at[c, pl.ds(st, RPT)])

    return agg_kernel(h2lay, src_pg, dst_pd)


def _agg2_call(h2, src_pg, dst_pd):
    F = C
    ew = EPAD // NW
    nch = ew // K
    cpr = F // L

    @functools.partial(
        pl.kernel,
        out_type=jax.ShapeDtypeStruct((NC, NPAD, F), jnp.float32),
        mesh=_mesh(),
        scratch_types=[
            pltpu.VMEM((K,), jnp.int32),
            pltpu.VMEM((K,), jnp.int32),
            pltpu.VMEM((K, F), jnp.float32),
            pltpu.VMEM((RPT, F), jnp.float32),
            pltpu.VMEM_SHARED((NPAD, F), jnp.float32),
            pltpu.SemaphoreType.DMA,
        ],
        compiler_params=pltpu.CompilerParams(use_tc_tiling_on_sc=False),
    )
    def agg_kernel(h_ref, src_ref, dst_ref, out_ref, sidx, didx, rows, zb, acc, sem):
        c = lax.axis_index("c")
        s = lax.axis_index("s")
        w = s * NC + c
        zero16 = jnp.zeros((L,), jnp.float32)

        def zb_body(j, carry):
            zb[j, pl.ds(0, L)] = zero16
            return carry

        lax.fori_loop(0, RPT * cpr, zb_body, 0)
        pltpu.sync_copy(zb, acc.at[pl.ds(pl.multiple_of(s * RPT, 8), RPT)])
        plsc.subcore_barrier()

        def body(i, carry):
            off = pl.multiple_of(w * ew + i * K, 8)
            pltpu.sync_copy(src_ref.at[pl.ds(off, K)], sidx)
            pltpu.sync_copy(dst_ref.at[pl.ds(off, K)], didx)
            pltpu.async_copy(h_ref.at[sidx], rows, sem).wait()
            pltpu.sync_copy(rows, acc.at[didx], add=True)
            return carry

        lax.fori_loop(0, nch, body, 0)
        plsc.subcore_barrier()
        st = pl.multiple_of(s * RPT, 8)
        pltpu.sync_copy(acc.at[pl.ds(st, RPT)], out_ref.at[c, pl.ds(st, RPT)])

    return agg_kernel(h2, src_pg, dst_pd)


def _norm_from(deg_row):
    return jnp.where(deg_row > 0.0, lax.rsqrt(deg_row), 0.0)


def _mm1_call(x, W1, degT):
    F = H // NC

    def body(x_ref, w_ref, deg_ref, o_ref):
        ns = _norm_from(deg_ref[:, 0])
        y = jnp.dot(x_ref[...], w_ref[...], preferred_element_type=jnp.float32)
        y = y * ns[:, None]
        o_ref[0] = y[:, :F]
        o_ref[1] = y[:, F:]

    return pl.pallas_call(
        body,
        grid=(N // BN,),
        in_specs=[
            pl.BlockSpec((BN, D), lambda i: (i, 0)),
            pl.BlockSpec((D, H), lambda i: (0, 0)),
            pl.BlockSpec((BN, 2), lambda i: (i, 0)),
        ],
        out_specs=pl.BlockSpec((2, BN, F), lambda i: (0, i, 0)),
        out_shape=jax.ShapeDtypeStruct((2, N, F), jnp.float32),
    )(x, W1, degT)


def _mid_call(parts, degT, b1, W2):
    F = H // NC

    def body(p_ref, deg_ref, b1_ref, w2_ref, o_ref):
        agg = jnp.concatenate([p_ref[0], p_ref[1]], axis=1)
        nd = _norm_from(deg_ref[:, 1])
        ns = _norm_from(deg_ref[:, 0])
        t = jnp.maximum(agg * nd[:, None] + b1_ref[...][None, :], 0.0) * ns[:, None]
        o_ref[...] = jnp.dot(t, w2_ref[...], preferred_element_type=jnp.float32)

    return pl.pallas_call(
        body,
        grid=(N // BN,),
        in_specs=[
            pl.BlockSpec((2, BN, F), lambda i: (0, i, 0)),
            pl.BlockSpec((BN, 2), lambda i: (i, 0)),
            pl.BlockSpec((H,), lambda i: (0,)),
            pl.BlockSpec((H, C), lambda i: (0, 0)),
        ],
        out_specs=pl.BlockSpec((BN, C), lambda i: (i, 0)),
        out_shape=jax.ShapeDtypeStruct((N, C), jnp.float32),
    )(parts, degT, b1, W2)


def _fin_call(parts2, degT, b2):
    def body(p_ref, deg_ref, b2_ref, o_ref):
        agg = p_ref[0] + p_ref[1]
        nd = _norm_from(deg_ref[:, 1])
        o_ref[...] = agg * nd[:, None] + b2_ref[...][None, :]

    return pl.pallas_call(
        body,
        grid=(N // BN,),
        in_specs=[
            pl.BlockSpec((2, BN, C), lambda i: (0, i, 0)),
            pl.BlockSpec((BN, 2), lambda i: (i, 0)),
            pl.BlockSpec((C,), lambda i: (0,)),
        ],
        out_specs=pl.BlockSpec((BN, C), lambda i: (i, 0)),
        out_shape=jax.ShapeDtypeStruct((N, C), jnp.float32),
    )(parts2, degT, b2)


def kernel(x, edge_index, W1, b1, W2, b2):
    src = edge_index[0]
    dst = edge_index[1]
    pad = jnp.arange(EPAD - E, dtype=jnp.int32)
    dst_pd = jnp.concatenate([dst, N + (pad % 16)])
    src_pd = jnp.concatenate([src, N + (pad % 16)])
    src_pg = jnp.concatenate([src, pad % N])

    deg = _deg_call(src_pd, dst_pd)
    degT = deg.T
    h = _mm1_call(x, W1, degT)
    parts1 = _agg_split_call(h, src_pg, dst_pd)
    h2 = _mid_call(parts1, degT, b1, W2)
    parts2 = _agg2_call(h2, src_pg, dst_pd)
    return _fin_call(parts2, degT, b2)


# speedup vs baseline: 16.6477x; 3.0641x over previous
import functools

import jax
import jax.numpy as jnp
from jax import lax
from jax.experimental import pallas as pl
from jax.experimental.pallas import tpu as pltpu
from jax.experimental.pallas import tpu_sc as plsc

N = 10000
D = 128
H = 128
C = 16
E = 320000
NC, NS, L = 2, 16, 16
NW = NC * NS
NPAD = 10240
RPT = NPAD // NS
EPAD = 327680
K = 128
BN = 1000


def _mesh():
    return plsc.VectorSubcoreMesh(
        core_axis_name="c", subcore_axis_name="s", num_cores=NC, num_subcores=NS
    )


def _deg_call(src2d, dst2d):
    nch = EPAD // K // NS

    @functools.partial(
        pl.kernel,
        out_type=jax.ShapeDtypeStruct((2, NPAD), jnp.float32),
        mesh=_mesh(),
        scratch_types=[
            pltpu.VMEM((nch, K), jnp.int32),
            pltpu.VMEM((K,), jnp.float32),
            pltpu.VMEM((RPT,), jnp.float32),
            pltpu.VMEM_SHARED((NPAD,), jnp.float32),
        ] + [pltpu.SemaphoreType.DMA] * 4,
    )
    def deg_kernel(src_ref, dst_ref, out_ref, idx2, ones, zb, acc, *sems):
        c = lax.axis_index("c")
        s = lax.axis_index("s")
        one16 = jnp.ones((L,), jnp.float32)
        zero16 = jnp.zeros((L,), jnp.float32)
        for j in range(K // L):
            ones[pl.ds(j * L, L)] = one16

        def zb_body(j, carry):
            zb[pl.ds(pl.multiple_of(j * L, 8), L)] = zero16
            return carry

        lax.fori_loop(0, RPT // L, zb_body, 0)
        pltpu.sync_copy(zb, acc.at[pl.ds(pl.multiple_of(s * RPT, 8), RPT)])

        @pl.when(c == 0)
        def _():
            pltpu.sync_copy(src_ref.at[pl.ds(s * nch, nch)], idx2)

        @pl.when(c == 1)
        def _():
            pltpu.sync_copy(dst_ref.at[pl.ds(s * nch, nch)], idx2)

        plsc.subcore_barrier()
        nb = len(sems)
        for b in range(nb):
            pltpu.async_copy(ones, acc.at[idx2.at[b]], sems[b], add=True)

        @pl.loop(0, nch, step=nb)
        def _(g0):
            for b in range(nb):
                i = g0 + b
                pltpu.make_async_copy(ones, acc.at[idx2.at[0]], sems[b]).wait()

                @pl.when(i + nb < nch)
                def _():
                    pltpu.async_copy(ones, acc.at[idx2.at[i + nb]], sems[b], add=True)

        plsc.subcore_barrier()
        st = pl.multiple_of(s * RPT, 8)
        pltpu.sync_copy(acc.at[pl.ds(st, RPT)], out_ref.at[c, pl.ds(st, RPT)])

    return deg_kernel(src2d, dst2d)


NB = 4
ZR = 160


def _zero_acc(zb, acc, s, cpr):
    zero16 = jnp.zeros((L,), jnp.float32)

    def zb_body(j, carry):
        zb[j // cpr, pl.ds((j % cpr) * L, L)] = zero16
        return carry

    lax.fori_loop(0, ZR * cpr, zb_body, 0)

    def zc_body(j, carry):
        pltpu.sync_copy(zb, acc.at[pl.ds(pl.multiple_of(s * RPT + j * ZR, 8), ZR)])
        return carry

    lax.fori_loop(0, RPT // ZR, zc_body, 0)


def _agg_ring(tab, sidx2, didx2, rows, acc, gsems, ssems, nch):
    for b in range(NB):
        pltpu.async_copy(tab.at[sidx2.at[b]], rows.at[b], gsems[b])

    @pl.loop(0, nch, step=NB)
    def _(g0):
        for b in range(NB):
            i = g0 + b
            pltpu.make_async_copy(tab.at[sidx2.at[0]], rows.at[b], gsems[b]).wait()
            pltpu.async_copy(rows.at[b], acc.at[didx2.at[i]], ssems[b], add=True)

            @pl.when(i + NB < nch)
            def _():
                pltpu.make_async_copy(rows.at[b], acc.at[didx2.at[0]], ssems[b]).wait()
                pltpu.async_copy(tab.at[sidx2.at[i + NB]], rows.at[b], gsems[b])

    for b in range(NB):
        pltpu.make_async_copy(rows.at[b], acc.at[didx2.at[0]], ssems[b]).wait()


def _agg_split_call(h2lay, src_pg, dst_pd):
    F = H // NC
    nch = EPAD // K // NS
    cpr = F // L

    @functools.partial(
        pl.kernel,
        out_type=jax.ShapeDtypeStruct((NC, NPAD, F), jnp.float32),
        mesh=_mesh(),
        scratch_types=[
            pltpu.VMEM((nch, K), jnp.int32),
            pltpu.VMEM((nch, K), jnp.int32),
            pltpu.VMEM((NB, K, F), jnp.float32),
            pltpu.VMEM((ZR, F), jnp.float32),
            pltpu.VMEM_SHARED((NPAD, F), jnp.float32),
        ] + [pltpu.SemaphoreType.DMA] * (2 * NB),
        compiler_params=pltpu.CompilerParams(use_tc_tiling_on_sc=False),
    )
    def agg_kernel(h_ref, src_ref, dst_ref, out_ref, sidx2, didx2, rows, zb, acc, *sems):
        c = lax.axis_index("c")
        s = lax.axis_index("s")
        _zero_acc(zb, acc, s, cpr)
        pltpu.sync_copy(src_ref.at[pl.ds(s * nch, nch)], sidx2)
        pltpu.sync_copy(dst_ref.at[pl.ds(s * nch, nch)], didx2)
        plsc.subcore_barrier()
        _agg_ring(h_ref.at[c], sidx2, didx2, rows, acc, sems[:NB], sems[NB:], nch)
        plsc.subcore_barrier()
        st = pl.multiple_of(s * RPT, 8)
        pltpu.sync_copy(acc.at[pl.ds(st, RPT)], out_ref.at[c, pl.ds(st, RPT)])

    return agg_kernel(h2lay, src_pg, dst_pd)


def _agg2_call(h2, src_pg, dst_pd):
    F = C
    nch = EPAD // K // NW
    cpr = F // L

    @functools.partial(
        pl.kernel,
        out_type=jax.ShapeDtypeStruct((NC, NPAD, F), jnp.float32),
        mesh=_mesh(),
        scratch_types=[
            pltpu.VMEM((nch, K), jnp.int32),
            pltpu.VMEM((nch, K), jnp.int32),
            pltpu.VMEM((NB, K, F), jnp.float32),
            pltpu.VMEM((ZR, F), jnp.float32),
            pltpu.VMEM_SHARED((NPAD, F), jnp.float32),
        ] + [pltpu.SemaphoreType.DMA] * (2 * NB),
        compiler_params=pltpu.CompilerParams(use_tc_tiling_on_sc=False),
    )
    def agg_kernel(h_ref, src_ref, dst_ref, out_ref, sidx2, didx2, rows, zb, acc, *sems):
        c = lax.axis_index("c")
        s = lax.axis_index("s")
        w = s * NC + c
        _zero_acc(zb, acc, s, cpr)
        pltpu.sync_copy(src_ref.at[pl.ds(w * nch, nch)], sidx2)
        pltpu.sync_copy(dst_ref.at[pl.ds(w * nch, nch)], didx2)
        plsc.subcore_barrier()
        _agg_ring(h_ref, sidx2, didx2, rows, acc, sems[:NB], sems[NB:], nch)
        plsc.subcore_barrier()
        st = pl.multiple_of(s * RPT, 8)
        pltpu.sync_copy(acc.at[pl.ds(st, RPT)], out_ref.at[c, pl.ds(st, RPT)])

    return agg_kernel(h2, src_pg, dst_pd)


def _norm_from(deg_row):
    return jnp.where(deg_row > 0.0, lax.rsqrt(deg_row), 0.0)


def _mm1_call(x, W1, degT):
    F = H // NC

    def body(x_ref, w_ref, deg_ref, o_ref):
        ns = _norm_from(deg_ref[:, 0])
        y = jnp.dot(x_ref[...], w_ref[...], preferred_element_type=jnp.float32)
        y = y * ns[:, None]
        o_ref[0] = y[:, :F]
        o_ref[1] = y[:, F:]

    return pl.pallas_call(
        body,
        grid=(N // BN,),
        in_specs=[
            pl.BlockSpec((BN, D), lambda i: (i, 0)),
            pl.BlockSpec((D, H), lambda i: (0, 0)),
            pl.BlockSpec((BN, 2), lambda i: (i, 0)),
        ],
        out_specs=pl.BlockSpec((2, BN, F), lambda i: (0, i, 0)),
        out_shape=jax.ShapeDtypeStruct((2, N, F), jnp.float32),
    )(x, W1, degT)


def _mid_call(parts, degT, b1, W2):
    F = H // NC

    def body(p_ref, deg_ref, b1_ref, w2_ref, o_ref):
        agg = jnp.concatenate([p_ref[0], p_ref[1]], axis=1)
        nd = _norm_from(deg_ref[:, 1])
        ns = _norm_from(deg_ref[:, 0])
        t = jnp.maximum(agg * nd[:, None] + b1_ref[...][None, :], 0.0) * ns[:, None]
        o_ref[...] = jnp.dot(t, w2_ref[...], preferred_element_type=jnp.float32)

    return pl.pallas_call(
        body,
        grid=(N // BN,),
        in_specs=[
            pl.BlockSpec((2, BN, F), lambda i: (0, i, 0)),
            pl.BlockSpec((BN, 2), lambda i: (i, 0)),
            pl.BlockSpec((H,), lambda i: (0,)),
            pl.BlockSpec((H, C), lambda i: (0, 0)),
        ],
        out_specs=pl.BlockSpec((BN, C), lambda i: (i, 0)),
        out_shape=jax.ShapeDtypeStruct((N, C), jnp.float32),
    )(parts, degT, b1, W2)


def _fin_call(parts2, degT, b2):
    def body(p_ref, deg_ref, b2_ref, o_ref):
        agg = p_ref[0] + p_ref[1]
        nd = _norm_from(deg_ref[:, 1])
        o_ref[...] = agg * nd[:, None] + b2_ref[...][None, :]

    return pl.pallas_call(
        body,
        grid=(N // BN,),
        in_specs=[
            pl.BlockSpec((2, BN, C), lambda i: (0, i, 0)),
            pl.BlockSpec((BN, 2), lambda i: (i, 0)),
            pl.BlockSpec((C,), lambda i: (0,)),
        ],
        out_specs=pl.BlockSpec((BN, C), lambda i: (i, 0)),
        out_shape=jax.ShapeDtypeStruct((N, C), jnp.float32),
    )(parts2, degT, b2)


def kernel(x, edge_index, W1, b1, W2, b2):
    src = edge_index[0]
    dst = edge_index[1]
    pad = jnp.arange(EPAD - E, dtype=jnp.int32)
    dst_pd = jnp.concatenate([dst, N + (pad % 16)]).reshape(EPAD // K, K)
    src_pd = jnp.concatenate([src, N + (pad % 16)]).reshape(EPAD // K, K)
    src_pg = jnp.concatenate([src, pad % N]).reshape(EPAD // K, K)

    deg = _deg_call(src_pd, dst_pd)
    degT = deg.T
    h = _mm1_call(x, W1, degT)
    parts1 = _agg_split_call(h, src_pg, dst_pd)
    h2 = _mid_call(parts1, degT, b1, W2)
    parts2 = _agg2_call(h2, src_pg, dst_pd)
    return _fin_call(parts2, degT, b2)


# speedup vs baseline: 16.8609x; 1.0128x over previous
import functools

import jax
import jax.numpy as jnp
from jax import lax
from jax.experimental import pallas as pl
from jax.experimental.pallas import tpu as pltpu
from jax.experimental.pallas import tpu_sc as plsc

N = 10000
D = 128
H = 128
C = 16
E = 320000
NC, NS, L = 2, 16, 16
NW = NC * NS
NPAD = 10240
RPT = NPAD // NS
EPAD = 327680
K = 128
BN = 1000


def _mesh():
    return plsc.VectorSubcoreMesh(
        core_axis_name="c", subcore_axis_name="s", num_cores=NC, num_subcores=NS
    )


def _deg_call(src2d, dst2d):
    nch = EPAD // K // NS
    nhalf = nch // 2

    @functools.partial(
        pl.kernel,
        out_type=jax.ShapeDtypeStruct((2, NPAD), jnp.float32),
        mesh=_mesh(),
        scratch_types=[
            pltpu.VMEM((nhalf, K), jnp.int32),
            pltpu.VMEM((K,), jnp.float32),
            pltpu.VMEM((RPT,), jnp.float32),
            pltpu.VMEM_SHARED((NPAD,), jnp.float32),
        ] + [pltpu.SemaphoreType.DMA] * 8,
    )
    def deg_kernel(src_ref, dst_ref, out_ref, idx2, ones, zb, acc, *sems):
        c = lax.axis_index("c")
        s = lax.axis_index("s")
        one16 = jnp.ones((L,), jnp.float32)
        zero16 = jnp.zeros((L,), jnp.float32)
        for j in range(K // L):
            ones[pl.ds(j * L, L)] = one16

        def zb_body(j, carry):
            zb[pl.ds(pl.multiple_of(j * L, 8), L)] = zero16
            return carry

        lax.fori_loop(0, RPT // L, zb_body, 0)
        pltpu.sync_copy(zb, acc.at[pl.ds(pl.multiple_of(s * RPT, 8), RPT)])
        plsc.subcore_barrier()
        nb = len(sems)

        def run(ref):
            for half in range(2):
                pltpu.sync_copy(ref.at[pl.ds(s * nch + half * nhalf, nhalf)], idx2)
                for b in range(nb):
                    pltpu.async_copy(ones, acc.at[idx2.at[b]], sems[b], add=True)

                @pl.loop(0, nhalf, step=nb)
                def _(g0):
                    for b in range(nb):
                        i = g0 + b
                        pltpu.make_async_copy(ones, acc.at[idx2.at[0]], sems[b]).wait()

                        @pl.when(i + nb < nhalf)
                        def _():
                            pltpu.async_copy(
                                ones, acc.at[idx2.at[i + nb]], sems[b], add=True
                            )

        @pl.when(c == 0)
        def _():
            run(src_ref)

        @pl.when(c == 1)
        def _():
            run(dst_ref)

        plsc.subcore_barrier()
        st = pl.multiple_of(s * RPT, 8)
        pltpu.sync_copy(acc.at[pl.ds(st, RPT)], out_ref.at[c, pl.ds(st, RPT)])

    return deg_kernel(src2d, dst2d)


NB = 5
ZR = 40


def _zero_acc(zb, acc, s, cpr):
    zero16 = jnp.zeros((L,), jnp.float32)

    def zb_body(j, carry):
        zb[j // cpr, pl.ds((j % cpr) * L, L)] = zero16
        return carry

    lax.fori_loop(0, ZR * cpr, zb_body, 0)

    def zc_body(j, carry):
        pltpu.sync_copy(zb, acc.at[pl.ds(pl.multiple_of(s * RPT + j * ZR, 8), ZR)])
        return carry

    lax.fori_loop(0, RPT // ZR, zc_body, 0)


def _agg_ring(tab, sidx2, didx2, rows, acc, gsems, ssems, nch):
    for b in range(NB):
        pltpu.async_copy(tab.at[sidx2.at[b]], rows.at[b], gsems[b])

    @pl.loop(0, nch, step=NB)
    def _(g0):
        for b in range(NB):
            i = g0 + b
            pltpu.make_async_copy(tab.at[sidx2.at[0]], rows.at[b], gsems[b]).wait()
            pltpu.async_copy(rows.at[b], acc.at[didx2.at[i]], ssems[b], add=True)

            @pl.when(i + NB < nch)
            def _():
                pltpu.make_async_copy(rows.at[b], acc.at[didx2.at[0]], ssems[b]).wait()
                pltpu.async_copy(tab.at[sidx2.at[i + NB]], rows.at[b], gsems[b])

    for b in range(NB):
        pltpu.make_async_copy(rows.at[b], acc.at[didx2.at[0]], ssems[b]).wait()


def _agg_split_call(h2lay, src_pg, dst_pd):
    F = H // NC
    nch = EPAD // K // NS
    cpr = F // L

    @functools.partial(
        pl.kernel,
        out_type=jax.ShapeDtypeStruct((NC, NPAD, F), jnp.float32),
        mesh=_mesh(),
        scratch_types=[
            pltpu.VMEM((nch, K), jnp.int32),
            pltpu.VMEM((nch, K), jnp.int32),
            pltpu.VMEM((NB, K, F), jnp.float32),
            pltpu.VMEM((ZR, F), jnp.float32),
            pltpu.VMEM_SHARED((NPAD, F), jnp.float32),
        ] + [pltpu.SemaphoreType.DMA] * (2 * NB),
        compiler_params=pltpu.CompilerParams(use_tc_tiling_on_sc=False),
    )
    def agg_kernel(h_ref, src_ref, dst_ref, out_ref, sidx2, didx2, rows, zb, acc, *sems):
        c = lax.axis_index("c")
        s = lax.axis_index("s")
        _zero_acc(zb, acc, s, cpr)
        pltpu.sync_copy(src_ref.at[pl.ds(s * nch, nch)], sidx2)
        pltpu.sync_copy(dst_ref.at[pl.ds(s * nch, nch)], didx2)
        plsc.subcore_barrier()
        _agg_ring(h_ref.at[c], sidx2, didx2, rows, acc, sems[:NB], sems[NB:], nch)
        plsc.subcore_barrier()
        st = pl.multiple_of(s * RPT, 8)
        pltpu.sync_copy(acc.at[pl.ds(st, RPT)], out_ref.at[c, pl.ds(st, RPT)])

    return agg_kernel(h2lay, src_pg, dst_pd)


def _agg2_call(h2, src_pg, dst_pd):
    F = C
    nch = EPAD // K // NW
    cpr = F // L

    @functools.partial(
        pl.kernel,
        out_type=jax.ShapeDtypeStruct((NC, NPAD, F), jnp.float32),
        mesh=_mesh(),
        scratch_types=[
            pltpu.VMEM((nch, K), jnp.int32),
            pltpu.VMEM((nch, K), jnp.int32),
            pltpu.VMEM((NB, K, F), jnp.float32),
            pltpu.VMEM((ZR, F), jnp.float32),
            pltpu.VMEM_SHARED((NPAD, F), jnp.float32),
        ] + [pltpu.SemaphoreType.DMA] * (2 * NB),
        compiler_params=pltpu.CompilerParams(use_tc_tiling_on_sc=False),
    )
    def agg_kernel(h_ref, src_ref, dst_ref, out_ref, sidx2, didx2, rows, zb, acc, *sems):
        c = lax.axis_index("c")
        s = lax.axis_index("s")
        w = s * NC + c
        _zero_acc(zb, acc, s, cpr)
        pltpu.sync_copy(src_ref.at[pl.ds(w * nch, nch)], sidx2)
        pltpu.sync_copy(dst_ref.at[pl.ds(w * nch, nch)], didx2)
        plsc.subcore_barrier()
        _agg_ring(h_ref, sidx2, didx2, rows, acc, sems[:NB], sems[NB:], nch)
        plsc.subcore_barrier()
        st = pl.multiple_of(s * RPT, 8)
        pltpu.sync_copy(acc.at[pl.ds(st, RPT)], out_ref.at[c, pl.ds(st, RPT)])

    return agg_kernel(h2, src_pg, dst_pd)


def _norm_from(deg_row):
    return jnp.where(deg_row > 0.0, lax.rsqrt(deg_row), 0.0)


def _mm1_call(x, W1, degT):
    F = H // NC

    def body(x_ref, w_ref, deg_ref, o_ref):
        ns = _norm_from(deg_ref[:, 0])
        y = jnp.dot(x_ref[...], w_ref[...], preferred_element_type=jnp.float32)
        y = y * ns[:, None]
        o_ref[0] = y[:, :F]
        o_ref[1] = y[:, F:]

    return pl.pallas_call(
        body,
        grid=(N // BN,),
        in_specs=[
            pl.BlockSpec((BN, D), lambda i: (i, 0)),
            pl.BlockSpec((D, H), lambda i: (0, 0)),
            pl.BlockSpec((BN, 2), lambda i: (i, 0)),
        ],
        out_specs=pl.BlockSpec((2, BN, F), lambda i: (0, i, 0)),
        out_shape=jax.ShapeDtypeStruct((2, N, F), jnp.float32),
    )(x, W1, degT)


def _mid_call(parts, degT, b1, W2):
    F = H // NC

    def body(p_ref, deg_ref, b1_ref, w2_ref, o_ref):
        agg = jnp.concatenate([p_ref[0], p_ref[1]], axis=1)
        nd = _norm_from(deg_ref[:, 1])
        ns = _norm_from(deg_ref[:, 0])
        t = jnp.maximum(agg * nd[:, None] + b1_ref[...][None, :], 0.0) * ns[:, None]
        o_ref[...] = jnp.dot(t, w2_ref[...], preferred_element_type=jnp.float32)

    return pl.pallas_call(
        body,
        grid=(N // BN,),
        in_specs=[
            pl.BlockSpec((2, BN, F), lambda i: (0, i, 0)),
            pl.BlockSpec((BN, 2), lambda i: (i, 0)),
            pl.BlockSpec((H,), lambda i: (0,)),
            pl.BlockSpec((H, C), lambda i: (0, 0)),
        ],
        out_specs=pl.BlockSpec((BN, C), lambda i: (i, 0)),
        out_shape=jax.ShapeDtypeStruct((N, C), jnp.float32),
    )(parts, degT, b1, W2)


def _fin_call(parts2, degT, b2):
    def body(p_ref, deg_ref, b2_ref, o_ref):
        agg = p_ref[0] + p_ref[1]
        nd = _norm_from(deg_ref[:, 1])
        o_ref[...] = agg * nd[:, None] + b2_ref[...][None, :]

    return pl.pallas_call(
        body,
        grid=(N // BN,),
        in_specs=[
            pl.BlockSpec((2, BN, C), lambda i: (0, i, 0)),
            pl.BlockSpec((BN, 2), lambda i: (i, 0)),
            pl.BlockSpec((C,), lambda i: (0,)),
        ],
        out_specs=pl.BlockSpec((BN, C), lambda i: (i, 0)),
        out_shape=jax.ShapeDtypeStruct((N, C), jnp.float32),
    )(parts2, degT, b2)


def kernel(x, edge_index, W1, b1, W2, b2):
    src = edge_index[0]
    dst = edge_index[1]
    pad = jnp.arange(EPAD - E, dtype=jnp.int32)
    dst_pd = jnp.concatenate([dst, N + (pad % 16)]).reshape(EPAD // K, K)
    src_pd = jnp.concatenate([src, N + (pad % 16)]).reshape(EPAD // K, K)
    src_pg = jnp.concatenate([src, pad % N]).reshape(EPAD // K, K)

    deg = _deg_call(src_pd, dst_pd)
    degT = deg.T
    h = _mm1_call(x, W1, degT)
    parts1 = _agg_split_call(h, src_pg, dst_pd)
    h2 = _mid_call(parts1, degT, b1, W2)
    parts2 = _agg2_call(h2, src_pg, dst_pd)
    return _fin_call(parts2, degT, b2)
